# Initial kernel scaffold; baseline (speedup 1.0000x reference)
import jax
import jax.numpy as jnp
from jax.experimental import pallas as pl


def kernel(features, xyz_0, xyz_1, xyz_2, xyz_3, neigh_idx_0, neigh_idx_1, neigh_idx_2, neigh_idx_3, sub_idx_0, sub_idx_1, sub_idx_2, sub_idx_3, interp_idx_0, interp_idx_1, interp_idx_2, interp_idx_3, labels, params):
    raise NotImplementedError("write your pallas kernel here")


# speedup vs baseline: 36.5254x; 36.5254x over previous
import jax
import jax.numpy as jnp
from jax import lax
from jax.experimental import pallas as pl
from jax.experimental.pallas import tpu as pltpu
from jax.experimental.pallas import tpu_sc as plsc

_INTERPRET = False
DIMS = [16, 64, 128, 256]
K = 16
NW = 32


def _p128(c):
    return -(-c // 128) * 128


def _p16(c):
    return -(-c // 16) * 16


def _rows(m, per_row, budget=1 << 20):
    t = m
    while t % 2 == 0 and (t // 2) % 8 == 0 and t * per_row > budget:
        t //= 2
    return t


def _act(y, act):
    if act == "relu":
        return jnp.maximum(y, 0.0)
    if act == "leaky":
        return jnp.where(y >= 0, y, 0.2 * y)
    return y


def _sc_gather(table, idx):
    n, c = table.shape
    m = idx.shape[0]
    per_w = m // NW
    limit = (240 * 1024) // (c * 4)
    ch = per_w
    nsplit = 1
    while ch > limit or ch % 8 != 0:
        nsplit += 1
        while per_w % nsplit:
            nsplit += 1
        ch = per_w // nsplit
    mesh = plsc.VectorSubcoreMesh(core_axis_name="c", subcore_axis_name="s")

    def body(table_hbm, idx_hbm, out_hbm, idx_v, rows_v, sem):
        wid = lax.axis_index("s") * 2 + lax.axis_index("c")
        base = wid * per_w

        def step(j, carry):
            off = base + j * ch
            pltpu.sync_copy(idx_hbm.at[pl.ds(off, ch)], idx_v)
            pltpu.async_copy(table_hbm.at[idx_v], rows_v, sem).wait()
            pltpu.sync_copy(rows_v, out_hbm.at[pl.ds(off, ch)])
            return carry

        lax.fori_loop(0, per_w // ch, step, 0)

    f = pl.kernel(
        body,
        out_type=jax.ShapeDtypeStruct((m, c), jnp.float32),
        mesh=mesh,
        compiler_params=pltpu.CompilerParams(use_tc_tiling_on_sc=False),
        scratch_types=[
            pltpu.VMEM((ch,), jnp.int32),
            pltpu.VMEM((ch, c), jnp.float32),
            pltpu.SemaphoreType.DMA,
        ],
    )
    return f(table, idx)


def _gather(table, idx):
    n, c = table.shape
    cp = _p16(c)
    if cp != c:
        table = jnp.pad(table, ((0, 0), (0, cp - c)))
    m0 = idx.shape[0]
    mp = -(-m0 // 256) * 256
    if mp != m0:
        idx = jnp.pad(idx, (0, mp - m0))
    return _sc_gather(table, idx)


def _mm(xs, wt, b):
    m = xs[0].shape[0]
    cout = wt.shape[1]
    per = sum(_p128(x.shape[1]) for x in xs) + _p128(cout)
    t = _rows(m, per)
    g = m // t
    nx = len(xs)

    def kern(*refs):
        xr = refs[:nx]
        wr = refs[nx]
        br = refs[nx + 1]
        y_ref = refs[nx + 2]
        if nx == 1:
            xcat = xr[0][...]
        else:
            xcat = jnp.concatenate([r[...] for r in xr], axis=1)
        y = jnp.dot(xcat, wr[...], preferred_element_type=jnp.float32)
        y_ref[...] = y + br[...]

    in_specs = (
        [pl.BlockSpec((t, x.shape[1]), lambda i: (i, 0)) for x in xs]
        + [pl.BlockSpec(wt.shape, lambda i: (0, 0))]
        + [pl.BlockSpec((1, cout), lambda i: (0, 0))]
    )
    return pl.pallas_call(
        kern,
        grid=(g,),
        in_specs=in_specs,
        out_specs=pl.BlockSpec((t, cout), lambda i: (i, 0)),
        out_shape=jax.ShapeDtypeStruct((m, cout), jnp.float32),
        interpret=_INTERPRET,
    )(*xs, wt, b)


def _colsum(y2d):
    m, c = y2d.shape
    t = _rows(m, _p128(c))
    g = m // t

    def kern(y_ref, s_ref):
        s = jnp.sum(y_ref[...], axis=0, keepdims=True)

        @pl.when(pl.program_id(0) == 0)
        def _():
            s_ref[...] = s

        @pl.when(pl.program_id(0) != 0)
        def _():
            s_ref[...] = s_ref[...] + s

    return pl.pallas_call(
        kern,
        grid=(g,),
        in_specs=[pl.BlockSpec((t, c), lambda i: (i, 0))],
        out_specs=pl.BlockSpec((1, c), lambda i: (0, 0)),
        out_shape=jax.ShapeDtypeStruct((1, c), jnp.float32),
        interpret=_INTERPRET,
    )(y2d)


def _colsqsum(y2d, mean):
    m, c = y2d.shape
    t = _rows(m, _p128(c))
    g = m // t

    def kern(y_ref, m_ref, s_ref):
        d = y_ref[...] - m_ref[...]
        s = jnp.sum(d * d, axis=0, keepdims=True)

        @pl.when(pl.program_id(0) == 0)
        def _():
            s_ref[...] = s

        @pl.when(pl.program_id(0) != 0)
        def _():
            s_ref[...] = s_ref[...] + s

    return pl.pallas_call(
        kern,
        grid=(g,),
        in_specs=[
            pl.BlockSpec((t, c), lambda i: (i, 0)),
            pl.BlockSpec((1, c), lambda i: (0, 0)),
        ],
        out_specs=pl.BlockSpec((1, c), lambda i: (0, 0)),
        out_shape=jax.ShapeDtypeStruct((1, c), jnp.float32),
        interpret=_INTERPRET,
    )(y2d, mean)


def _normalize(yp, mean, var, gamma, beta, act):
    m, c = yp.shape
    t = _rows(m, 2 * _p128(c))
    g = m // t

    def kern(y_ref, m_ref, v_ref, g_ref, b_ref, o_ref):
        den = jnp.sqrt(v_ref[...] + 1e-5)
        y = (y_ref[...] - m_ref[...]) / den
        o_ref[...] = _act(y * g_ref[...] + b_ref[...], act)

    vec = lambda: pl.BlockSpec((1, c), lambda i: (0, 0))
    return pl.pallas_call(
        kern,
        grid=(g,),
        in_specs=[pl.BlockSpec((t, c), lambda i: (i, 0)), vec(), vec(), vec(), vec()],
        out_specs=pl.BlockSpec((t, c), lambda i: (i, 0)),
        out_shape=jax.ShapeDtypeStruct((m, c), jnp.float32),
        interpret=_INTERPRET,
    )(yp, mean, var, gamma, beta)


def _bn_stats(yp):
    m, c = yp.shape
    if c < 128 and (m * c) % 128 == 0:
        gcols = 128 // c
        flat = yp.reshape(m * c // 128, 128)
        cs = _colsum(flat)
        chan_sum = jnp.sum(cs.reshape(gcols, c), axis=0, keepdims=True)
        mean = chan_sum / m
        mean128 = jnp.tile(mean, (1, gcols))
        sq = _colsqsum(flat, mean128)
        var = jnp.sum(sq.reshape(gcols, c), axis=0, keepdims=True) / m
    else:
        cs = _colsum(yp)
        mean = cs / m
        var = _colsqsum(yp, mean) / m
    return mean, var


def _conv_bn(xs, p, act="relu"):
    wt = jnp.transpose(p["W"])
    b = p["b"].reshape(1, -1)
    count = 1
    for s in xs[0].shape[:-1]:
        count *= s
    flat = [x.reshape(count, x.shape[-1]) for x in xs]
    yp = _mm(flat, wt, b)
    mean, var = _bn_stats(yp)
    return _normalize(yp, mean, var, p["gamma"].reshape(1, -1),
                      p["beta"].reshape(1, -1), act)


def _relpos_mm(xyz_i, g3, p, ca):
    n, k, cp = g3.shape
    wt = jnp.transpose(p["W"])
    d2 = wt.shape[1]
    b = p["b"].reshape(1, -1)
    per = k * (_p128(cp) + _p128(ca) + _p128(d2) + 2 * _p128(10)) + _p128(3)
    t = _rows(n, per)
    g = n // t

    def kern(xyz_ref, g_ref, w_ref, b_ref, fn_ref, y_ref):
        gg = g_ref[...]
        nx = gg[:, :, 0:3]
        fn_ref[...] = gg[:, :, 3 : 3 + ca]
        xt = jnp.broadcast_to(xyz_ref[...][:, None, :], (t, k, 3))
        rel = xt - nx
        r2 = rel * rel
        ss = (r2[:, :, 0:1] + r2[:, :, 1:2]) + r2[:, :, 2:3]
        dis = jnp.sqrt(ss + 1e-12)
        feat = jnp.concatenate([dis, rel, xt, nx], axis=2)
        y = jnp.dot(feat.reshape(t * k, 10), w_ref[...], preferred_element_type=jnp.float32)
        y = y + b_ref[...]
        y_ref[...] = y.reshape(t, k, d2)

    fn, yp = pl.pallas_call(
        kern,
        grid=(g,),
        in_specs=[
            pl.BlockSpec((t, 3), lambda i: (i, 0)),
            pl.BlockSpec((t, k, cp), lambda i: (i, 0, 0)),
            pl.BlockSpec((10, d2), lambda i: (0, 0)),
            pl.BlockSpec((1, d2), lambda i: (0, 0)),
        ],
        out_specs=[
            pl.BlockSpec((t, k, ca), lambda i: (i, 0, 0)),
            pl.BlockSpec((t, k, d2), lambda i: (i, 0, 0)),
        ],
        out_shape=[
            jax.ShapeDtypeStruct((n, k, ca), jnp.float32),
            jax.ShapeDtypeStruct((n, k, d2), jnp.float32),
        ],
        interpret=_INTERPRET,
    )(xyz_i, g3, wt, b)
    return fn, yp


def _attpool_mm(fa, ca, fb, fc_p, mlp_p):
    n, k, cap = fa.shape
    cb = fb.shape[2]
    d = ca + cb
    wfc = jnp.transpose(fc_p["W"])
    wm = jnp.transpose(mlp_p["W"])
    dm = wm.shape[1]
    bm = mlp_p["b"].reshape(1, -1)
    per = k * (_p128(cap) + _p128(cb) + 3 * _p128(d)) + 2 * _p128(dm)
    t = _rows(n, per)
    g = n // t

    def kern(fa_ref, fb_ref, wfc_ref, wm_ref, bm_ref, y_ref):
        fsa = fa_ref[...][:, :, :ca]
        fsb = fb_ref[...]
        fs = jnp.concatenate([fsa, fsb], axis=2)
        att = jnp.dot(fs.reshape(t * k, d), wfc_ref[...], preferred_element_type=jnp.float32)
        att = att.reshape(t, k, d)
        mx = jnp.max(att, axis=1, keepdims=True)
        e = jnp.exp(att - mx)
        sc = e / jnp.sum(e, axis=1, keepdims=True)
        agg = jnp.sum(fs * sc, axis=1)
        y_ref[...] = jnp.dot(agg, wm_ref[...], preferred_element_type=jnp.float32) + bm_ref[...]

    return pl.pallas_call(
        kern,
        grid=(g,),
        in_specs=[
            pl.BlockSpec((t, k, cap), lambda i: (i, 0, 0)),
            pl.BlockSpec((t, k, cb), lambda i: (i, 0, 0)),
            pl.BlockSpec((d, d), lambda i: (0, 0)),
            pl.BlockSpec((d, dm), lambda i: (0, 0)),
            pl.BlockSpec((1, dm), lambda i: (0, 0)),
        ],
        out_specs=pl.BlockSpec((t, dm), lambda i: (i, 0)),
        out_shape=jax.ShapeDtypeStruct((n, dm), jnp.float32),
        interpret=_INTERPRET,
    )(fa, fb, wfc, wm, bm)


def _merge_leaky(yp1, mv1, p1, yp2, mv2, p2):
    m, c = yp1.shape
    t = _rows(m, 3 * _p128(c))
    g = m // t

    def kern(y1_ref, m1_ref, v1_ref, g1_ref, b1_ref,
             y2_ref, m2_ref, v2_ref, g2_ref, b2_ref, o_ref):
        def bn(y_ref, m_ref, v_ref, g_ref, b_ref):
            y = (y_ref[...] - m_ref[...]) / jnp.sqrt(v_ref[...] + 1e-5)
            return y * g_ref[...] + b_ref[...]

        y = bn(y1_ref, m1_ref, v1_ref, g1_ref, b1_ref) + bn(y2_ref, m2_ref, v2_ref, g2_ref, b2_ref)
        o_ref[...] = _act(y, "leaky")

    vec = lambda: pl.BlockSpec((1, c), lambda i: (0, 0))
    big = lambda: pl.BlockSpec((t, c), lambda i: (i, 0))
    return pl.pallas_call(
        kern,
        grid=(g,),
        in_specs=[big(), vec(), vec(), vec(), vec(), big(), vec(), vec(), vec(), vec()],
        out_specs=big(),
        out_shape=jax.ShapeDtypeStruct((m, c), jnp.float32),
        interpret=_INTERPRET,
    )(
        yp1, mv1[0], mv1[1], p1["gamma"].reshape(1, -1), p1["beta"].reshape(1, -1),
        yp2, mv2[0], mv2[1], p2["gamma"].reshape(1, -1), p2["beta"].reshape(1, -1),
    )


def _maxpool_k(g3):
    n, k, c = g3.shape
    t = _rows(n, k * _p128(c) + _p128(c))
    g = n // t

    def kern(g_ref, o_ref):
        o_ref[...] = jnp.max(g_ref[...], axis=1)

    return pl.pallas_call(
        kern,
        grid=(g,),
        in_specs=[pl.BlockSpec((t, k, c), lambda i: (i, 0, 0))],
        out_specs=pl.BlockSpec((t, c), lambda i: (i, 0)),
        out_shape=jax.ShapeDtypeStruct((n, c), jnp.float32),
        interpret=_INTERPRET,
    )(g3)


def _final_logits(x, p):
    m, ci = x.shape
    wt = jnp.transpose(p["W"])
    nc = wt.shape[1]
    b = p["b"].reshape(1, -1)
    t = _rows(m, 4 * 128)
    g = m // t

    def kern(x_ref, w_ref, b_ref, lo_ref, am_ref):
        y = jnp.dot(x_ref[...], w_ref[...], preferred_element_type=jnp.float32) + b_ref[...]
        lo_ref[...] = y
        ids = lax.broadcasted_iota(jnp.int32, (t, nc), 1)
        mx = jnp.max(y, axis=1, keepdims=True)
        am = jnp.min(jnp.where(y == mx, ids, nc), axis=1)
        am_ref[...] = am.reshape(t, 1)

    return pl.pallas_call(
        kern,
        grid=(g,),
        in_specs=[
            pl.BlockSpec((t, ci), lambda i: (i, 0)),
            pl.BlockSpec((ci, nc), lambda i: (0, 0)),
            pl.BlockSpec((1, nc), lambda i: (0, 0)),
        ],
        out_specs=[
            pl.BlockSpec((t, nc), lambda i: (i, 0)),
            pl.BlockSpec((t, 1), lambda i: (i, 0)),
        ],
        out_shape=[
            jax.ShapeDtypeStruct((m, nc), jnp.float32),
            jax.ShapeDtypeStruct((m, 1), jnp.int32),
        ],
        interpret=_INTERPRET,
    )(x, wt, b)


def _bn_apply_flat(yp3, p, act):
    n, k, c = yp3.shape
    flat = yp3.reshape(n * k, c)
    mean, var = _bn_stats(flat)
    out = _normalize(flat, mean, var, p["gamma"].reshape(1, -1),
                     p["beta"].reshape(1, -1), act)
    return out.reshape(n, k, c)


def _lfa(L, f, xyz_i, neigh_i, d):
    n = f.shape[0]
    d2 = d // 2
    f_pc = _conv_bn([f], L["mlp1"])
    tbl = jnp.concatenate([xyz_i, f_pc], axis=1)
    cp = _p16(3 + d2)
    g1 = _gather(tbl, neigh_i).reshape(n, K, cp)
    f_neigh, yp3 = _relpos_mm(xyz_i, g1, L["bb_mlp1"], d2)
    f_xyz1 = _bn_apply_flat(yp3, L["bb_mlp1"], "relu")
    yp = _attpool_mm(f_neigh, d2, f_xyz1, L["att1_fc"], L["att1_mlp"])
    mean, var = _bn_stats(yp)
    f_agg = _normalize(yp, mean, var, L["att1_mlp"]["gamma"].reshape(1, -1),
                       L["att1_mlp"]["beta"].reshape(1, -1), "relu")
    f_xyz2 = _conv_bn([f_xyz1.reshape(n * K, d2)], L["bb_mlp2"]).reshape(n, K, d2)
    cp2 = _p16(d2)
    g2 = _gather(f_agg, neigh_i).reshape(n, K, cp2)
    yp = _attpool_mm(g2, d2, f_xyz2, L["att2_fc"], L["att2_mlp"])
    mean, var = _bn_stats(yp)
    att2 = _normalize(yp, mean, var, L["att2_mlp"]["gamma"].reshape(1, -1),
                      L["att2_mlp"]["beta"].reshape(1, -1), "relu")
    yp1 = _mm([att2], jnp.transpose(L["mlp2"]["W"]), L["mlp2"]["b"].reshape(1, -1))
    yp2 = _mm([f], jnp.transpose(L["shortcut"]["W"]), L["shortcut"]["b"].reshape(1, -1))
    return _merge_leaky(yp1, _bn_stats(yp1), L["mlp2"], yp2, _bn_stats(yp2), L["shortcut"])


def kernel(features, xyz_0, xyz_1, xyz_2, xyz_3,
           neigh_idx_0, neigh_idx_1, neigh_idx_2, neigh_idx_3,
           sub_idx_0, sub_idx_1, sub_idx_2, sub_idx_3,
           interp_idx_0, interp_idx_1, interp_idx_2, interp_idx_3,
           labels, params):
    P = params
    xyz = [xyz_0[0], xyz_1[0], xyz_2[0], xyz_3[0]]
    neigh = [a[0].reshape(-1).astype(jnp.int32)
             for a in (neigh_idx_0, neigh_idx_1, neigh_idx_2, neigh_idx_3)]
    sub = [a[0].reshape(-1).astype(jnp.int32)
           for a in (sub_idx_0, sub_idx_1, sub_idx_2, sub_idx_3)]
    interp = [a[0].reshape(-1).astype(jnp.int32)
              for a in (interp_idx_0, interp_idx_1, interp_idx_2, interp_idx_3)]

    f = _conv_bn([jnp.transpose(features[0])], P["fc_start"])
    enc = []
    for i in range(4):
        x = _lfa(P["encoder"][i], f, xyz[i], neigh[i], DIMS[i])
        c = x.shape[1]
        n_next = sub[i].shape[0] // K
        g3 = _gather(x, sub[i]).reshape(n_next, K, c)
        f = _maxpool_k(g3)
        if i == 0:
            enc.append(x)
        enc.append(f)

    f = _conv_bn([enc[-1]], P["dc_start"])
    for j in range(4):
        ii = interp[3 - j]
        up = ii.shape[0]
        fi = _gather(f, ii)[:up]
        f = _conv_bn([enc[-j - 2], fi], P["decoder"][j])
    f = _conv_bn([f], P["fc_end"][0])
    f = _conv_bn([f], P["fc_end"][1])
    logits_rm, am = _final_logits(f, P["fc_end"][2])

    logits = jnp.transpose(logits_rm)[None]
    predicts = am.reshape(1, -1)
    return logits, labels[..., 0], predicts


# speedup vs baseline: 36.5855x; 1.0016x over previous
import jax
import jax.numpy as jnp
from jax import lax
from jax.experimental import pallas as pl
from jax.experimental.pallas import tpu as pltpu
from jax.experimental.pallas import tpu_sc as plsc

_INTERPRET = False
DIMS = [16, 64, 128, 256]
K = 16
NW = 32


def _p128(c):
    return -(-c // 128) * 128


def _p16(c):
    return -(-c // 16) * 16


def _rows(m, per_row, budget=1 << 20):
    t = m
    while t % 2 == 0 and (t // 2) % 8 == 0 and t * per_row > budget:
        t //= 2
    return t


def _stride_sum_k(v):
    s = 8
    while s >= 1:
        v = v[:, 0:s] + v[:, s : 2 * s]
        s //= 2
    return v


def _act(y, act):
    if act == "relu":
        return jnp.maximum(y, 0.0)
    if act == "leaky":
        return jnp.where(y >= 0, y, 0.2 * y)
    return y


def _sc_gather(table, idx):
    n, c = table.shape
    m = idx.shape[0]
    per_w = m // NW
    limit = (240 * 1024) // (c * 4)
    ch = per_w
    nsplit = 1
    while ch > limit or ch % 8 != 0:
        nsplit += 1
        while per_w % nsplit:
            nsplit += 1
        ch = per_w // nsplit
    mesh = plsc.VectorSubcoreMesh(core_axis_name="c", subcore_axis_name="s")

    def body(table_hbm, idx_hbm, out_hbm, idx_v, rows_v, sem):
        wid = lax.axis_index("s") * 2 + lax.axis_index("c")
        base = wid * per_w

        def step(j, carry):
            off = base + j * ch
            pltpu.sync_copy(idx_hbm.at[pl.ds(off, ch)], idx_v)
            pltpu.async_copy(table_hbm.at[idx_v], rows_v, sem).wait()
            pltpu.sync_copy(rows_v, out_hbm.at[pl.ds(off, ch)])
            return carry

        lax.fori_loop(0, per_w // ch, step, 0)

    f = pl.kernel(
        body,
        out_type=jax.ShapeDtypeStruct((m, c), jnp.float32),
        mesh=mesh,
        compiler_params=pltpu.CompilerParams(use_tc_tiling_on_sc=False),
        scratch_types=[
            pltpu.VMEM((ch,), jnp.int32),
            pltpu.VMEM((ch, c), jnp.float32),
            pltpu.SemaphoreType.DMA,
        ],
    )
    return f(table, idx)


def _gather(table, idx):
    n, c = table.shape
    cp = _p16(c)
    if cp != c:
        table = jnp.pad(table, ((0, 0), (0, cp - c)))
    m0 = idx.shape[0]
    mp = -(-m0 // 256) * 256
    if mp != m0:
        idx = jnp.pad(idx, (0, mp - m0))
    return _sc_gather(table, idx)


def _mm(xs, wt, b):
    m = xs[0].shape[0]
    cout = wt.shape[1]
    per = sum(_p128(x.shape[1]) for x in xs) + _p128(cout)
    t = _rows(m, per)
    g = m // t
    nx = len(xs)

    def kern(*refs):
        xr = refs[:nx]
        wr = refs[nx]
        br = refs[nx + 1]
        y_ref = refs[nx + 2]
        if nx == 1:
            xcat = xr[0][...]
        else:
            xcat = jnp.concatenate([r[...] for r in xr], axis=1)
        y = jnp.dot(xcat, wr[...], preferred_element_type=jnp.float32)
        y_ref[...] = y + br[...]

    in_specs = (
        [pl.BlockSpec((t, x.shape[1]), lambda i: (i, 0)) for x in xs]
        + [pl.BlockSpec(wt.shape, lambda i: (0, 0))]
        + [pl.BlockSpec((1, cout), lambda i: (0, 0))]
    )
    return pl.pallas_call(
        kern,
        grid=(g,),
        in_specs=in_specs,
        out_specs=pl.BlockSpec((t, cout), lambda i: (i, 0)),
        out_shape=jax.ShapeDtypeStruct((m, cout), jnp.float32),
        interpret=_INTERPRET,
    )(*xs, wt, b)


def _colsum(y2d):
    m, c = y2d.shape
    t = _rows(m, _p128(c))
    g = m // t

    def kern(y_ref, s_ref):
        s = jnp.sum(y_ref[...], axis=0, keepdims=True)

        @pl.when(pl.program_id(0) == 0)
        def _():
            s_ref[...] = s

        @pl.when(pl.program_id(0) != 0)
        def _():
            s_ref[...] = s_ref[...] + s

    return pl.pallas_call(
        kern,
        grid=(g,),
        in_specs=[pl.BlockSpec((t, c), lambda i: (i, 0))],
        out_specs=pl.BlockSpec((1, c), lambda i: (0, 0)),
        out_shape=jax.ShapeDtypeStruct((1, c), jnp.float32),
        interpret=_INTERPRET,
    )(y2d)


def _colsqsum(y2d, mean):
    m, c = y2d.shape
    t = _rows(m, _p128(c))
    g = m // t

    def kern(y_ref, m_ref, s_ref):
        d = y_ref[...] - m_ref[...]
        s = jnp.sum(d * d, axis=0, keepdims=True)

        @pl.when(pl.program_id(0) == 0)
        def _():
            s_ref[...] = s

        @pl.when(pl.program_id(0) != 0)
        def _():
            s_ref[...] = s_ref[...] + s

    return pl.pallas_call(
        kern,
        grid=(g,),
        in_specs=[
            pl.BlockSpec((t, c), lambda i: (i, 0)),
            pl.BlockSpec((1, c), lambda i: (0, 0)),
        ],
        out_specs=pl.BlockSpec((1, c), lambda i: (0, 0)),
        out_shape=jax.ShapeDtypeStruct((1, c), jnp.float32),
        interpret=_INTERPRET,
    )(y2d, mean)


def _normalize(yp, mean, var, gamma, beta, act):
    m, c = yp.shape
    t = _rows(m, 2 * _p128(c))
    g = m // t

    def kern(y_ref, m_ref, v_ref, g_ref, b_ref, o_ref):
        den = jnp.sqrt(v_ref[...] + 1e-5)
        y = (y_ref[...] - m_ref[...]) / den
        o_ref[...] = _act(y * g_ref[...] + b_ref[...], act)

    vec = lambda: pl.BlockSpec((1, c), lambda i: (0, 0))
    return pl.pallas_call(
        kern,
        grid=(g,),
        in_specs=[pl.BlockSpec((t, c), lambda i: (i, 0)), vec(), vec(), vec(), vec()],
        out_specs=pl.BlockSpec((t, c), lambda i: (i, 0)),
        out_shape=jax.ShapeDtypeStruct((m, c), jnp.float32),
        interpret=_INTERPRET,
    )(yp, mean, var, gamma, beta)


def _bn_stats(yp):
    m, c = yp.shape
    if c < 128 and (m * c) % 128 == 0:
        gcols = 128 // c
        flat = yp.reshape(m * c // 128, 128)
        cs = _colsum(flat)
        chan_sum = jnp.sum(cs.reshape(gcols, c), axis=0, keepdims=True)
        mean = chan_sum / m
        mean128 = jnp.tile(mean, (1, gcols))
        sq = _colsqsum(flat, mean128)
        var = jnp.sum(sq.reshape(gcols, c), axis=0, keepdims=True) / m
    else:
        cs = _colsum(yp)
        mean = cs / m
        var = _colsqsum(yp, mean) / m
    return mean, var


def _conv_bn(xs, p, act="relu"):
    wt = jnp.transpose(p["W"])
    b = p["b"].reshape(1, -1)
    count = 1
    for s in xs[0].shape[:-1]:
        count *= s
    flat = [x.reshape(count, x.shape[-1]) for x in xs]
    yp = _mm(flat, wt, b)
    mean, var = _bn_stats(yp)
    return _normalize(yp, mean, var, p["gamma"].reshape(1, -1),
                      p["beta"].reshape(1, -1), act)


def _relpos_mm(xyz_i, g3, p, ca):
    n, k, cp = g3.shape
    wt = jnp.transpose(p["W"])
    d2 = wt.shape[1]
    b = p["b"].reshape(1, -1)
    per = k * (_p128(cp) + _p128(ca) + _p128(d2) + 2 * _p128(10)) + _p128(3)
    t = _rows(n, per)
    g = n // t

    def kern(xyz_ref, g_ref, w_ref, b_ref, fn_ref, y_ref):
        gg = g_ref[...]
        nx = gg[:, :, 0:3]
        fn_ref[...] = gg[:, :, 3 : 3 + ca]
        xt = jnp.broadcast_to(xyz_ref[...][:, None, :], (t, k, 3))
        rel = xt - nx
        r2 = rel * rel
        ss = (r2[:, :, 0:1] + r2[:, :, 2:3]) + r2[:, :, 1:2]
        dis = jnp.sqrt(ss + 1e-12)
        feat = jnp.concatenate([dis, rel, xt, nx], axis=2)
        y = jnp.dot(feat.reshape(t * k, 10), w_ref[...], preferred_element_type=jnp.float32)
        y = y + b_ref[...]
        y_ref[...] = y.reshape(t, k, d2)

    fn, yp = pl.pallas_call(
        kern,
        grid=(g,),
        in_specs=[
            pl.BlockSpec((t, 3), lambda i: (i, 0)),
            pl.BlockSpec((t, k, cp), lambda i: (i, 0, 0)),
            pl.BlockSpec((10, d2), lambda i: (0, 0)),
            pl.BlockSpec((1, d2), lambda i: (0, 0)),
        ],
        out_specs=[
            pl.BlockSpec((t, k, ca), lambda i: (i, 0, 0)),
            pl.BlockSpec((t, k, d2), lambda i: (i, 0, 0)),
        ],
        out_shape=[
            jax.ShapeDtypeStruct((n, k, ca), jnp.float32),
            jax.ShapeDtypeStruct((n, k, d2), jnp.float32),
        ],
        interpret=_INTERPRET,
    )(xyz_i, g3, wt, b)
    return fn, yp


def _attpool_mm(fa, ca, fb, fc_p, mlp_p):
    n, k, cap = fa.shape
    cb = fb.shape[2]
    d = ca + cb
    wfc = jnp.transpose(fc_p["W"])
    wm = jnp.transpose(mlp_p["W"])
    dm = wm.shape[1]
    bm = mlp_p["b"].reshape(1, -1)
    per = k * (_p128(cap) + _p128(cb) + 3 * _p128(d)) + 2 * _p128(dm)
    t = _rows(n, per)
    g = n // t

    def kern(fa_ref, fb_ref, wfc_ref, wm_ref, bm_ref, y_ref):
        fsa = fa_ref[...][:, :, :ca]
        fsb = fb_ref[...]
        fs = jnp.concatenate([fsa, fsb], axis=2)
        att = jnp.dot(fs.reshape(t * k, d), wfc_ref[...], preferred_element_type=jnp.float32)
        att = att.reshape(t, k, d)
        mx = jnp.max(att, axis=1, keepdims=True)
        e = jnp.exp(att - mx)
        sc = e / _stride_sum_k(e)
        agg = _stride_sum_k(fs * sc).reshape(t, d)
        y_ref[...] = jnp.dot(agg, wm_ref[...], preferred_element_type=jnp.float32) + bm_ref[...]

    return pl.pallas_call(
        kern,
        grid=(g,),
        in_specs=[
            pl.BlockSpec((t, k, cap), lambda i: (i, 0, 0)),
            pl.BlockSpec((t, k, cb), lambda i: (i, 0, 0)),
            pl.BlockSpec((d, d), lambda i: (0, 0)),
            pl.BlockSpec((d, dm), lambda i: (0, 0)),
            pl.BlockSpec((1, dm), lambda i: (0, 0)),
        ],
        out_specs=pl.BlockSpec((t, dm), lambda i: (i, 0)),
        out_shape=jax.ShapeDtypeStruct((n, dm), jnp.float32),
        interpret=_INTERPRET,
    )(fa, fb, wfc, wm, bm)


def _merge_leaky(yp1, mv1, p1, yp2, mv2, p2):
    m, c = yp1.shape
    t = _rows(m, 3 * _p128(c))
    g = m // t

    def kern(y1_ref, m1_ref, v1_ref, g1_ref, b1_ref,
             y2_ref, m2_ref, v2_ref, g2_ref, b2_ref, o_ref):
        def bn(y_ref, m_ref, v_ref, g_ref, b_ref):
            y = (y_ref[...] - m_ref[...]) / jnp.sqrt(v_ref[...] + 1e-5)
            return y * g_ref[...] + b_ref[...]

        y = bn(y1_ref, m1_ref, v1_ref, g1_ref, b1_ref) + bn(y2_ref, m2_ref, v2_ref, g2_ref, b2_ref)
        o_ref[...] = _act(y, "leaky")

    vec = lambda: pl.BlockSpec((1, c), lambda i: (0, 0))
    big = lambda: pl.BlockSpec((t, c), lambda i: (i, 0))
    return pl.pallas_call(
        kern,
        grid=(g,),
        in_specs=[big(), vec(), vec(), vec(), vec(), big(), vec(), vec(), vec(), vec()],
        out_specs=big(),
        out_shape=jax.ShapeDtypeStruct((m, c), jnp.float32),
        interpret=_INTERPRET,
    )(
        yp1, mv1[0], mv1[1], p1["gamma"].reshape(1, -1), p1["beta"].reshape(1, -1),
        yp2, mv2[0], mv2[1], p2["gamma"].reshape(1, -1), p2["beta"].reshape(1, -1),
    )


def _maxpool_k(g3):
    n, k, c = g3.shape
    t = _rows(n, k * _p128(c) + _p128(c))
    g = n // t

    def kern(g_ref, o_ref):
        o_ref[...] = jnp.max(g_ref[...], axis=1)

    return pl.pallas_call(
        kern,
        grid=(g,),
        in_specs=[pl.BlockSpec((t, k, c), lambda i: (i, 0, 0))],
        out_specs=pl.BlockSpec((t, c), lambda i: (i, 0)),
        out_shape=jax.ShapeDtypeStruct((n, c), jnp.float32),
        interpret=_INTERPRET,
    )(g3)


def _final_logits(x, p):
    m, ci = x.shape
    wt = jnp.transpose(p["W"])
    nc = wt.shape[1]
    b = p["b"].reshape(1, -1)
    t = _rows(m, 4 * 128)
    g = m // t

    def kern(x_ref, w_ref, b_ref, lo_ref, am_ref):
        y = jnp.dot(x_ref[...], w_ref[...], preferred_element_type=jnp.float32) + b_ref[...]
        lo_ref[...] = y
        ids = lax.broadcasted_iota(jnp.int32, (t, nc), 1)
        mx = jnp.max(y, axis=1, keepdims=True)
        am = jnp.min(jnp.where(y == mx, ids, nc), axis=1)
        am_ref[...] = am.reshape(t, 1)

    return pl.pallas_call(
        kern,
        grid=(g,),
        in_specs=[
            pl.BlockSpec((t, ci), lambda i: (i, 0)),
            pl.BlockSpec((ci, nc), lambda i: (0, 0)),
            pl.BlockSpec((1, nc), lambda i: (0, 0)),
        ],
        out_specs=[
            pl.BlockSpec((t, nc), lambda i: (i, 0)),
            pl.BlockSpec((t, 1), lambda i: (i, 0)),
        ],
        out_shape=[
            jax.ShapeDtypeStruct((m, nc), jnp.float32),
            jax.ShapeDtypeStruct((m, 1), jnp.int32),
        ],
        interpret=_INTERPRET,
    )(x, wt, b)


def _bn_apply_flat(yp3, p, act):
    n, k, c = yp3.shape
    flat = yp3.reshape(n * k, c)
    mean, var = _bn_stats(flat)
    out = _normalize(flat, mean, var, p["gamma"].reshape(1, -1),
                     p["beta"].reshape(1, -1), act)
    return out.reshape(n, k, c)


def _lfa(L, f, xyz_i, neigh_i, d):
    n = f.shape[0]
    d2 = d // 2
    f_pc = _conv_bn([f], L["mlp1"])
    tbl = jnp.concatenate([xyz_i, f_pc], axis=1)
    cp = _p16(3 + d2)
    g1 = _gather(tbl, neigh_i).reshape(n, K, cp)
    f_neigh, yp3 = _relpos_mm(xyz_i, g1, L["bb_mlp1"], d2)
    f_xyz1 = _bn_apply_flat(yp3, L["bb_mlp1"], "relu")
    yp = _attpool_mm(f_neigh, d2, f_xyz1, L["att1_fc"], L["att1_mlp"])
    mean, var = _bn_stats(yp)
    f_agg = _normalize(yp, mean, var, L["att1_mlp"]["gamma"].reshape(1, -1),
                       L["att1_mlp"]["beta"].reshape(1, -1), "relu")
    f_xyz2 = _conv_bn([f_xyz1.reshape(n * K, d2)], L["bb_mlp2"]).reshape(n, K, d2)
    cp2 = _p16(d2)
    g2 = _gather(f_agg, neigh_i).reshape(n, K, cp2)
    yp = _attpool_mm(g2, d2, f_xyz2, L["att2_fc"], L["att2_mlp"])
    mean, var = _bn_stats(yp)
    att2 = _normalize(yp, mean, var, L["att2_mlp"]["gamma"].reshape(1, -1),
                      L["att2_mlp"]["beta"].reshape(1, -1), "relu")
    yp1 = _mm([att2], jnp.transpose(L["mlp2"]["W"]), L["mlp2"]["b"].reshape(1, -1))
    yp2 = _mm([f], jnp.transpose(L["shortcut"]["W"]), L["shortcut"]["b"].reshape(1, -1))
    return _merge_leaky(yp1, _bn_stats(yp1), L["mlp2"], yp2, _bn_stats(yp2), L["shortcut"])


def kernel(features, xyz_0, xyz_1, xyz_2, xyz_3,
           neigh_idx_0, neigh_idx_1, neigh_idx_2, neigh_idx_3,
           sub_idx_0, sub_idx_1, sub_idx_2, sub_idx_3,
           interp_idx_0, interp_idx_1, interp_idx_2, interp_idx_3,
           labels, params):
    P = params
    xyz = [xyz_0[0], xyz_1[0], xyz_2[0], xyz_3[0]]
    neigh = [a[0].reshape(-1).astype(jnp.int32)
             for a in (neigh_idx_0, neigh_idx_1, neigh_idx_2, neigh_idx_3)]
    sub = [a[0].reshape(-1).astype(jnp.int32)
           for a in (sub_idx_0, sub_idx_1, sub_idx_2, sub_idx_3)]
    interp = [a[0].reshape(-1).astype(jnp.int32)
              for a in (interp_idx_0, interp_idx_1, interp_idx_2, interp_idx_3)]

    f = _conv_bn([jnp.transpose(features[0])], P["fc_start"])
    enc = []
    for i in range(4):
        x = _lfa(P["encoder"][i], f, xyz[i], neigh[i], DIMS[i])
        c = x.shape[1]
        n_next = sub[i].shape[0] // K
        g3 = _gather(x, sub[i]).reshape(n_next, K, c)
        f = _maxpool_k(g3)
        if i == 0:
            enc.append(x)
        enc.append(f)

    f = _conv_bn([enc[-1]], P["dc_start"])
    for j in range(4):
        ii = interp[3 - j]
        up = ii.shape[0]
        fi = _gather(f, ii)[:up]
        f = _conv_bn([enc[-j - 2], fi], P["decoder"][j])
    f = _conv_bn([f], P["fc_end"][0])
    f = _conv_bn([f], P["fc_end"][1])
    logits_rm, am = _final_logits(f, P["fc_end"][2])

    logits = jnp.transpose(logits_rm)[None]
    predicts = am.reshape(1, -1)
    return logits, labels[..., 0], predicts
